# baseline (speedup 1.0000x reference)
import jax
import jax.numpy as jnp
from jax import lax
from jax.experimental import pallas as pl
from jax.experimental.pallas import tpu as pltpu
from jax.experimental.pallas import tpu_sc as plsc

NUM_CANVASES = 3
B, C, H, W = 16, 3, 512, 512

_ROWS = B * C * H
_NC, _NS = 2, 16
_NW = _NC * _NS
_ROWS_PER_W = _ROWS // _NW
_NSPLIT = 4


def _sc_copy_body(src, dst, sems):
    wid = lax.axis_index("s") * _NC + lax.axis_index("c")
    base = wid * _ROWS_PER_W
    step = _ROWS_PER_W // _NSPLIT
    descs = []
    for j in range(_NSPLIT):
        sl = pl.ds(base + j * step, step)
        descs.append(pltpu.async_copy(src.at[sl], dst.at[sl], sems.at[j]))
    for d in descs:
        d.wait()


_sc_copy = pl.kernel(
    _sc_copy_body,
    out_type=jax.ShapeDtypeStruct((_ROWS, W), jnp.float32),
    mesh=plsc.VectorSubcoreMesh(
        core_axis_name="c", subcore_axis_name="s", num_cores=_NC, num_subcores=_NS
    ),
    scratch_types=[
        pltpu.SemaphoreType.DMA((_NSPLIT,)),
    ],
)


def kernel(img_batch, canvases):
    del canvases
    flat = img_batch.reshape(_ROWS, W)
    return _sc_copy(flat).reshape(B, C, H, W)


# speedup vs baseline: 28.8293x; 28.8293x over previous
import jax
import jax.numpy as jnp
from jax import lax
from jax.experimental import pallas as pl
from jax.experimental.pallas import tpu as pltpu
from jax.experimental.pallas import tpu_sc as plsc

NUM_CANVASES = 3
B, C, H, W = 16, 3, 512, 512

_ROWS = B * C * H
_S = 12288

_NC, _NS = 2, 16
_NW = _NC * _NS
_SC_PER_W = _S // _NW
_SC_CH = 48
_SC_NCH = _SC_PER_W // _SC_CH
_SC_NBUF = 4
_SC_OLAG = 2

_TC_BLOCK = 3072


def _ring_copy(src, dst, bufs, isems, osems, base, ch, nch, nbuf, olag):
    def start_in(j):
        b = j % nbuf
        return pltpu.async_copy(
            src.at[pl.ds(base + j * ch, ch)], bufs.at[b], isems.at[b]
        )

    def start_out(j):
        b = j % nbuf
        return pltpu.async_copy(
            bufs.at[b], dst.at[pl.ds(base + j * ch, ch)], osems.at[b]
        )

    descs = {}
    for j in range(min(nbuf, nch)):
        descs[("i", j)] = start_in(j)
    for i in range(nch):
        descs[("i", i)].wait()
        descs[("o", i)] = start_out(i)
        j = i - olag + nbuf
        if i >= olag and j < nch:
            descs[("o", i - olag)].wait()
            descs[("i", j)] = start_in(j)
    for i in range(max(nch - nbuf, 0), nch):
        descs[("o", i)].wait()


def _sc_body(src, out, bufs, isems, osems):
    wid = lax.axis_index("s") * _NC + lax.axis_index("c")
    base = wid * _SC_PER_W
    _ring_copy(src, out, bufs, isems, osems, base, _SC_CH, _SC_NCH, _SC_NBUF, _SC_OLAG)


_sc_store = pl.kernel(
    _sc_body,
    out_type=jax.ShapeDtypeStruct((_ROWS, W), jnp.float32),
    mesh=plsc.VectorSubcoreMesh(
        core_axis_name="c", subcore_axis_name="s", num_cores=_NC, num_subcores=_NS
    ),
    scratch_types=[
        pltpu.VMEM((_SC_NBUF, _SC_CH, W), jnp.float32),
        pltpu.SemaphoreType.DMA((_SC_NBUF,)),
        pltpu.SemaphoreType.DMA((_SC_NBUF,)),
    ],
)


def _tc_fill_body(src_ref, partial_ref, out_ref):
    del partial_ref
    out_ref[...] = src_ref[...]


def _tc_fill(flat, partial):
    nblk = (_ROWS - _S) // _TC_BLOCK
    return pl.pallas_call(
        _tc_fill_body,
        grid=(nblk,),
        in_specs=[
            pl.BlockSpec((_TC_BLOCK, W), lambda i: (_S // _TC_BLOCK + i, 0)),
            pl.BlockSpec(memory_space=pl.ANY),
        ],
        out_specs=pl.BlockSpec((_TC_BLOCK, W), lambda i: (_S // _TC_BLOCK + i, 0)),
        out_shape=jax.ShapeDtypeStruct((_ROWS, W), jnp.float32),
        input_output_aliases={1: 0},
    )(flat, partial)


def kernel(img_batch, canvases):
    del canvases
    flat = img_batch.reshape(_ROWS, W)
    partial = _sc_store(flat)
    return _tc_fill(flat, partial).reshape(B, C, H, W)


# speedup vs baseline: 29.0618x; 1.0081x over previous
import jax
import jax.numpy as jnp
from jax import lax
from jax.experimental import pallas as pl
from jax.experimental.pallas import tpu as pltpu
from jax.experimental.pallas import tpu_sc as plsc

NUM_CANVASES = 3
B, C, H, W = 16, 3, 512, 512

_ROWS = B * C * H
_S = 12288

_NC, _NS = 2, 16
_NW = _NC * _NS
_SC_PER_W = _S // _NW
_SC_CH = 48
_SC_NCH = _SC_PER_W // _SC_CH
_SC_NBUF = 4
_SC_OLAG = 2

_TC_BLOCK = 4096


def _ring_copy(src, dst, bufs, isems, osems, base, ch, nch, nbuf, olag):
    def start_in(j):
        b = j % nbuf
        return pltpu.async_copy(
            src.at[pl.ds(base + j * ch, ch)], bufs.at[b], isems.at[b]
        )

    def start_out(j):
        b = j % nbuf
        return pltpu.async_copy(
            bufs.at[b], dst.at[pl.ds(base + j * ch, ch)], osems.at[b]
        )

    descs = {}
    for j in range(min(nbuf, nch)):
        descs[("i", j)] = start_in(j)
    for i in range(nch):
        descs[("i", i)].wait()
        descs[("o", i)] = start_out(i)
        j = i - olag + nbuf
        if i >= olag and j < nch:
            descs[("o", i - olag)].wait()
            descs[("i", j)] = start_in(j)
    for i in range(max(nch - nbuf, 0), nch):
        descs[("o", i)].wait()


def _sc_body(src, out, bufs, isems, osems):
    wid = lax.axis_index("s") * _NC + lax.axis_index("c")
    base = wid * _SC_PER_W
    _ring_copy(src, out, bufs, isems, osems, base, _SC_CH, _SC_NCH, _SC_NBUF, _SC_OLAG)


_sc_store = pl.kernel(
    _sc_body,
    out_type=jax.ShapeDtypeStruct((_ROWS, W), jnp.float32),
    mesh=plsc.VectorSubcoreMesh(
        core_axis_name="c", subcore_axis_name="s", num_cores=_NC, num_subcores=_NS
    ),
    scratch_types=[
        pltpu.VMEM((_SC_NBUF, _SC_CH, W), jnp.float32),
        pltpu.SemaphoreType.DMA((_SC_NBUF,)),
        pltpu.SemaphoreType.DMA((_SC_NBUF,)),
    ],
)


def _tc_fill_body(src_ref, partial_ref, out_ref):
    del partial_ref
    out_ref[...] = src_ref[...]


def _tc_fill(flat, partial):
    nblk = (_ROWS - _S) // _TC_BLOCK
    return pl.pallas_call(
        _tc_fill_body,
        grid=(nblk,),
        in_specs=[
            pl.BlockSpec((_TC_BLOCK, W), lambda i: (_S // _TC_BLOCK + i, 0)),
            pl.BlockSpec(memory_space=pl.ANY),
        ],
        out_specs=pl.BlockSpec((_TC_BLOCK, W), lambda i: (_S // _TC_BLOCK + i, 0)),
        out_shape=jax.ShapeDtypeStruct((_ROWS, W), jnp.float32),
        input_output_aliases={1: 0},
    )(flat, partial)


def kernel(img_batch, canvases):
    del canvases
    flat = img_batch.reshape(_ROWS, W)
    partial = _sc_store(flat)
    return _tc_fill(flat, partial).reshape(B, C, H, W)


# speedup vs baseline: 29.1853x; 1.0042x over previous
import jax
import jax.numpy as jnp
from jax import lax
from jax.experimental import pallas as pl
from jax.experimental.pallas import tpu as pltpu
from jax.experimental.pallas import tpu_sc as plsc

NUM_CANVASES = 3
B, C, H, W = 16, 3, 512, 512

_ROWS = B * C * H
_S = 12288

_NC, _NS = 2, 16
_NW = _NC * _NS
_SC_PER_W = _S // _NW
_SC_CH = 48
_SC_NCH = _SC_PER_W // _SC_CH
_SC_NBUF = 5
_SC_OLAG = 3

_TC_BLOCK = 4096


def _ring_copy(src, dst, bufs, isems, osems, base, ch, nch, nbuf, olag):
    def start_in(j):
        b = j % nbuf
        return pltpu.async_copy(
            src.at[pl.ds(base + j * ch, ch)], bufs.at[b], isems.at[b]
        )

    def start_out(j):
        b = j % nbuf
        return pltpu.async_copy(
            bufs.at[b], dst.at[pl.ds(base + j * ch, ch)], osems.at[b]
        )

    descs = {}
    for j in range(min(nbuf, nch)):
        descs[("i", j)] = start_in(j)
    for i in range(nch):
        descs[("i", i)].wait()
        descs[("o", i)] = start_out(i)
        j = i - olag + nbuf
        if i >= olag and j < nch:
            descs[("o", i - olag)].wait()
            descs[("i", j)] = start_in(j)
    for i in range(max(nch - nbuf, 0), nch):
        descs[("o", i)].wait()


def _sc_body(src, out, bufs, isems, osems):
    wid = lax.axis_index("s") * _NC + lax.axis_index("c")
    base = wid * _SC_PER_W
    _ring_copy(src, out, bufs, isems, osems, base, _SC_CH, _SC_NCH, _SC_NBUF, _SC_OLAG)


_sc_store = pl.kernel(
    _sc_body,
    out_type=jax.ShapeDtypeStruct((_ROWS, W), jnp.float32),
    mesh=plsc.VectorSubcoreMesh(
        core_axis_name="c", subcore_axis_name="s", num_cores=_NC, num_subcores=_NS
    ),
    scratch_types=[
        pltpu.VMEM((_SC_NBUF, _SC_CH, W), jnp.float32),
        pltpu.SemaphoreType.DMA((_SC_NBUF,)),
        pltpu.SemaphoreType.DMA((_SC_NBUF,)),
    ],
)


def _tc_fill_body(src_ref, partial_ref, out_ref):
    del partial_ref
    out_ref[...] = src_ref[...]


def _tc_fill(flat, partial):
    nblk = (_ROWS - _S) // _TC_BLOCK
    return pl.pallas_call(
        _tc_fill_body,
        grid=(nblk,),
        in_specs=[
            pl.BlockSpec((_TC_BLOCK, W), lambda i: (_S // _TC_BLOCK + i, 0)),
            pl.BlockSpec(memory_space=pl.ANY),
        ],
        out_specs=pl.BlockSpec((_TC_BLOCK, W), lambda i: (_S // _TC_BLOCK + i, 0)),
        out_shape=jax.ShapeDtypeStruct((_ROWS, W), jnp.float32),
        input_output_aliases={1: 0},
    )(flat, partial)


def kernel(img_batch, canvases):
    del canvases
    flat = img_batch.reshape(_ROWS, W)
    partial = _sc_store(flat)
    return _tc_fill(flat, partial).reshape(B, C, H, W)
